# baseline (speedup 1.0000x reference)
import functools

import jax
import jax.numpy as jnp
from jax import lax
from jax.experimental import pallas as pl
from jax.experimental.pallas import tpu as pltpu
from jax.experimental.pallas import tpu_sc as plsc

_NC = 2
_NS = 16
_LANES = 16
_CH = 80
_NBUF = 4
_NSLOT = 2 * _NBUF


def _make_deg_kernel(E, NPAD):
    NW = _NC * _NS
    EPW = E // NW
    ITERS = EPW // _CH
    SPT = NPAD // _NS
    DEPTH = 4
    mesh = plsc.VectorSubcoreMesh(core_axis_name="c", subcore_axis_name="s")

    @functools.partial(
        pl.kernel,
        out_type=jax.ShapeDtypeStruct((_NC, NPAD), jnp.float32),
        mesh=mesh,
        scratch_types=[
            pltpu.VMEM_SHARED((NPAD,), jnp.float32),
            pltpu.VMEM((EPW,), jnp.int32),
            pltpu.VMEM((ITERS, _CH), jnp.int32),
            pltpu.VMEM((_CH,), jnp.float32),
            pltpu.VMEM((SPT,), jnp.float32),
            pltpu.SemaphoreType.DMA,
        ],
    )
    def deg_kernel(col_hbm, parts_hbm, deg_sp, flat_v, cidx_v, ones_v,
                   buf_v, sem):
        c = lax.axis_index("c")
        s = lax.axis_index("s")
        wid = c * _NS + s

        def zb(i, carry):
            buf_v[pl.ds(i * _LANES, _LANES)] = jnp.zeros((_LANES,), jnp.float32)
            return carry

        lax.fori_loop(0, SPT // _LANES, zb, 0)
        pltpu.sync_copy(buf_v, deg_sp.at[pl.ds(s * SPT, SPT)])

        def ob(i, carry):
            ones_v[pl.ds(i * _LANES, _LANES)] = jnp.ones((_LANES,), jnp.float32)
            return carry

        lax.fori_loop(0, _CH // _LANES, ob, 0)
        pltpu.sync_copy(col_hbm.at[pl.ds(wid * EPW, EPW)], flat_v)
        PER_ROW = _CH // _LANES

        def rp(i, carry):
            v = flat_v[pl.ds(i * _LANES, _LANES)]
            cidx_v[i // PER_ROW,
                   pl.ds(lax.rem(i, PER_ROW) * _LANES, _LANES)] = v
            return carry

        lax.fori_loop(0, EPW // _LANES, rp, 0)
        plsc.subcore_barrier()

        def fire(j, carry):
            pltpu.async_copy(ones_v, deg_sp.at[cidx_v.at[j]], sem, add=True)

            @pl.when(j >= DEPTH)
            def _():
                pltpu.make_async_copy(ones_v, deg_sp.at[cidx_v.at[0]],
                                      sem).wait()

            return carry

        lax.fori_loop(0, ITERS, fire, 0)

        def drain(j, carry):
            pltpu.make_async_copy(ones_v, deg_sp.at[cidx_v.at[0]], sem).wait()
            return carry

        lax.fori_loop(0, DEPTH, drain, 0)
        plsc.subcore_barrier()

        pltpu.sync_copy(deg_sp.at[pl.ds(s * SPT, SPT)],
                        parts_hbm.at[c, pl.ds(s * SPT, SPT)])

    return deg_kernel


def _make_mm_kernel(N, NPAD, D):
    BLK = 2000
    GRID = N // BLK

    def body(x_ref, w_ref, deg_ref, h_ref):
        h = lax.dot_general(
            x_ref[...], w_ref[...], (((1,), (1,)), ((), ())),
            preferred_element_type=jnp.float32,
        )
        deg = deg_ref[:, 0:1] + deg_ref[:, 1:2]
        h_ref[...] = h * lax.rsqrt(deg + 1.0)

    return pl.pallas_call(
        body,
        grid=(GRID,),
        in_specs=[
            pl.BlockSpec((BLK, D), lambda i: (i, 0)),
            pl.BlockSpec((D, D), lambda i: (0, 0)),
            pl.BlockSpec((BLK, _NC), lambda i: (i, 0)),
        ],
        out_specs=pl.BlockSpec((BLK, D), lambda i: (i, 0)),
        out_shape=jax.ShapeDtypeStruct((N, D), jnp.float32),
    )


def _make_edge_kernel(E, N, NPAD, D):
    NW = _NC * _NS
    EPW = E // NW
    ITERS = EPW // _CH
    SPT = NPAD // _NS
    mesh = plsc.VectorSubcoreMesh(core_axis_name="c", subcore_axis_name="s")

    @functools.partial(
        pl.kernel,
        out_type=(
            jax.ShapeDtypeStruct((NPAD, D), jnp.float32),
            jax.ShapeDtypeStruct((NPAD, D), jnp.float32),
        ),
        mesh=mesh,
        scratch_types=[
            pltpu.VMEM_SHARED((NPAD, D), jnp.float32),
            pltpu.VMEM((_NSLOT, 2, _CH), jnp.int32),
        ] + [pltpu.VMEM((_CH, D), jnp.float32)] * _NBUF
          + [pltpu.SemaphoreType.DMA] * (2 * _NBUF + _NSLOT),
    )
    def edge_kernel(edge_hbm, h_hbm, p0_hbm, p1_hbm,
                    acc_sp, slots, *bufsem):
        bufs = bufsem[:_NBUF]
        gsem = bufsem[_NBUF:2 * _NBUF]
        ssem = bufsem[2 * _NBUF:3 * _NBUF]
        isem = bufsem[3 * _NBUF:]
        c = lax.axis_index("c")
        s = lax.axis_index("s")
        wid = c * _NS + s
        ebase = wid * EPW

        def load_idx(j, q):
            pltpu.async_copy(edge_hbm.at[pl.ds(ebase + j * _CH, _CH)],
                             slots.at[q, 0], isem[q])
            pltpu.async_copy(edge_hbm.at[pl.ds(E + ebase + j * _CH, _CH)],
                             slots.at[q, 1], isem[q])

        def wait_idx(q):
            pltpu.make_async_copy(edge_hbm.at[pl.ds(0, _CH)],
                                  slots.at[q, 0], isem[q]).wait()
            pltpu.make_async_copy(edge_hbm.at[pl.ds(0, _CH)],
                                  slots.at[q, 1], isem[q]).wait()

        def gather(b, q):
            pltpu.async_copy(h_hbm.at[slots.at[q, 0]], bufs[b], gsem[b])

        def scatter(b, q):
            pltpu.async_copy(bufs[b], acc_sp.at[slots.at[q, 1]], ssem[b],
                             add=True)

        def wait_g(b):
            pltpu.make_async_copy(h_hbm.at[slots.at[0, 0]], bufs[b],
                                  gsem[b]).wait()

        def wait_s(b):
            pltpu.make_async_copy(bufs[b], acc_sp.at[slots.at[0, 1]],
                                  ssem[b]).wait()

        for q in range(_NSLOT):
            load_idx(q, q)

        nchunks = jnp.maximum(
            0, jnp.minimum(SPT, jnp.int32(N) - s * SPT)) // _CH

        @pl.when(c == 0)
        def _ih():
            def hcp(t, carry):
                r0 = s * SPT + t * _CH
                pltpu.sync_copy(h_hbm.at[pl.ds(r0, _CH)],
                                acc_sp.at[pl.ds(r0, _CH)])
                return carry

            lax.fori_loop(0, nchunks, hcp, 0)

        @pl.when(c == 1)
        def _iz():
            def zb(i, carry):
                def zl(k, carry2):
                    bufs[0][i, pl.ds(k * _LANES, _LANES)] = jnp.zeros(
                        (_LANES,), jnp.float32)
                    return carry2

                lax.fori_loop(0, D // _LANES, zl, 0)
                return carry

            lax.fori_loop(0, _CH, zb, 0)

            def zcp(t, carry):
                pltpu.sync_copy(bufs[0],
                                acc_sp.at[pl.ds(s * SPT + t * _CH, _CH)])
                return carry

            lax.fori_loop(0, nchunks, zcp, 0)

        for b in range(_NBUF):
            wait_idx(2 * b)
            gather(b, 2 * b)
        plsc.subcore_barrier()

        def body(t, carry):
            for u in range(2 * _NBUF):
                b = u % _NBUF
                par = u // _NBUF
                q = 2 * b + par
                qn = 2 * b + (1 - par)
                j = 2 * _NBUF * t + u

                @pl.when(j < ITERS)
                def _(b=b, j=j, q=q, qn=qn):
                    wait_g(b)
                    scatter(b, q)

                    @pl.when(j + _NBUF < ITERS)
                    def _():
                        wait_s(b)

                        @pl.when(j + 2 * _NBUF < ITERS)
                        def _():
                            load_idx(j + 2 * _NBUF, q)

                        wait_idx(qn)
                        gather(b, qn)

            return carry

        lax.fori_loop(0, (ITERS + 2 * _NBUF - 1) // (2 * _NBUF), body, 0)
        for b in range(_NBUF):
            wait_s(b)

        plsc.subcore_barrier()

        @pl.when(c == 0)
        def _w0():
            pltpu.sync_copy(acc_sp.at[pl.ds(s * SPT, SPT)],
                            p0_hbm.at[pl.ds(s * SPT, SPT)])

        @pl.when(c == 1)
        def _w1():
            pltpu.sync_copy(acc_sp.at[pl.ds(s * SPT, SPT)],
                            p1_hbm.at[pl.ds(s * SPT, SPT)])

    return edge_kernel


def _make_final_kernel(N, NPAD, D):
    BLK = 2000
    GRID = N // BLK

    def body(p0_ref, p1_ref, deg_ref, out_ref):
        deg = deg_ref[:, 0:1] + deg_ref[:, 1:2]
        dis = lax.rsqrt(deg + 1.0)
        z = (p0_ref[...] + p1_ref[...]) * dis
        out_ref[...] = jnp.where(z > 0, 2.0 * z, z)

    return pl.pallas_call(
        body,
        grid=(GRID,),
        in_specs=[
            pl.BlockSpec((BLK, D), lambda i: (i, 0)),
            pl.BlockSpec((BLK, D), lambda i: (i, 0)),
            pl.BlockSpec((BLK, _NC), lambda i: (i, 0)),
        ],
        out_specs=pl.BlockSpec((BLK, D), lambda i: (i, 0)),
        out_shape=jax.ShapeDtypeStruct((N, D), jnp.float32),
    )


def kernel(x, edge_index, edge_attr, W):
    N, D = x.shape
    E = edge_index.shape[1]
    NPAD = ((N + 1023) // 1024) * 1024

    eflat = edge_index.reshape(2 * E)
    parts = _make_deg_kernel(E, NPAD)(edge_index[1])
    parts_t = parts.T
    h = _make_mm_kernel(N, NPAD, D)(x, W, parts_t)
    p0, p1 = _make_edge_kernel(E, N, NPAD, D)(eflat, h)
    return _make_final_kernel(N, NPAD, D)(p0, p1, parts_t)


# speedup vs baseline: 1.0720x; 1.0720x over previous
import functools

import jax
import jax.numpy as jnp
from jax import lax
from jax.experimental import pallas as pl
from jax.experimental.pallas import tpu as pltpu
from jax.experimental.pallas import tpu_sc as plsc

_NC = 2
_NS = 16
_LANES = 16
_CH = 80
_NBUF = 4
_NSLOT = 2 * _NBUF


def _make_deg_kernel(E, NPAD):
    NW = _NC * _NS
    EPW = E // NW
    ITERS = EPW // _CH
    SPT = NPAD // _NS
    DEPTH = 4
    mesh = plsc.VectorSubcoreMesh(core_axis_name="c", subcore_axis_name="s")

    @functools.partial(
        pl.kernel,
        out_type=jax.ShapeDtypeStruct((_NC, NPAD), jnp.float32),
        mesh=mesh,
        scratch_types=[
            pltpu.VMEM_SHARED((NPAD,), jnp.float32),
            pltpu.VMEM((EPW,), jnp.int32),
            pltpu.VMEM((ITERS, _CH), jnp.int32),
            pltpu.VMEM((_CH,), jnp.float32),
            pltpu.VMEM((SPT,), jnp.float32),
            pltpu.SemaphoreType.DMA,
        ],
    )
    def deg_kernel(col_hbm, parts_hbm, deg_sp, flat_v, cidx_v, ones_v,
                   buf_v, sem):
        c = lax.axis_index("c")
        s = lax.axis_index("s")
        wid = c * _NS + s

        def zb(i, carry):
            buf_v[pl.ds(i * _LANES, _LANES)] = jnp.zeros((_LANES,), jnp.float32)
            return carry

        lax.fori_loop(0, SPT // _LANES, zb, 0)
        pltpu.sync_copy(buf_v, deg_sp.at[pl.ds(s * SPT, SPT)])

        def ob(i, carry):
            ones_v[pl.ds(i * _LANES, _LANES)] = jnp.ones((_LANES,), jnp.float32)
            return carry

        lax.fori_loop(0, _CH // _LANES, ob, 0)
        pltpu.sync_copy(col_hbm.at[pl.ds(E + wid * EPW, EPW)], flat_v)
        PER_ROW = _CH // _LANES

        def rp(i, carry):
            v = flat_v[pl.ds(i * _LANES, _LANES)]
            cidx_v[i // PER_ROW,
                   pl.ds(lax.rem(i, PER_ROW) * _LANES, _LANES)] = v
            return carry

        lax.fori_loop(0, EPW // _LANES, rp, 0)
        plsc.subcore_barrier()

        def fire(j, carry):
            pltpu.async_copy(ones_v, deg_sp.at[cidx_v.at[j]], sem, add=True)

            @pl.when(j >= DEPTH)
            def _():
                pltpu.make_async_copy(ones_v, deg_sp.at[cidx_v.at[0]],
                                      sem).wait()

            return carry

        lax.fori_loop(0, ITERS, fire, 0)

        def drain(j, carry):
            pltpu.make_async_copy(ones_v, deg_sp.at[cidx_v.at[0]], sem).wait()
            return carry

        lax.fori_loop(0, DEPTH, drain, 0)
        plsc.subcore_barrier()

        pltpu.sync_copy(deg_sp.at[pl.ds(s * SPT, SPT)],
                        parts_hbm.at[c, pl.ds(s * SPT, SPT)])

    return deg_kernel


def _make_mm_kernel(N, NPAD, D):
    BLK = 2000
    GRID = N // BLK

    def body(x_ref, w_ref, deg_ref, h_ref):
        h = lax.dot_general(
            x_ref[...], w_ref[...], (((1,), (1,)), ((), ())),
            preferred_element_type=jnp.float32,
        )
        deg = deg_ref[:, 0:1] + deg_ref[:, 1:2]
        h_ref[...] = h * lax.rsqrt(deg + 1.0)

    return pl.pallas_call(
        body,
        grid=(GRID,),
        in_specs=[
            pl.BlockSpec((BLK, D), lambda i: (i, 0)),
            pl.BlockSpec((D, D), lambda i: (0, 0)),
            pl.BlockSpec((BLK, _NC), lambda i: (i, 0)),
        ],
        out_specs=pl.BlockSpec((BLK, D), lambda i: (i, 0)),
        out_shape=jax.ShapeDtypeStruct((N, D), jnp.float32),
    )


def _make_edge_kernel(E, N, NPAD, D):
    NW = _NC * _NS
    EPW = E // NW
    ITERS = EPW // _CH
    SPT = NPAD // _NS
    mesh = plsc.VectorSubcoreMesh(core_axis_name="c", subcore_axis_name="s")

    @functools.partial(
        pl.kernel,
        out_type=(
            jax.ShapeDtypeStruct((NPAD, D), jnp.float32),
            jax.ShapeDtypeStruct((NPAD, D), jnp.float32),
        ),
        mesh=mesh,
        scratch_types=[
            pltpu.VMEM_SHARED((NPAD, D), jnp.float32),
            pltpu.VMEM((_NSLOT, 2, _CH), jnp.int32),
        ] + [pltpu.VMEM((_CH, D), jnp.float32)] * _NBUF
          + [pltpu.SemaphoreType.DMA] * (2 * _NBUF + _NSLOT),
    )
    def edge_kernel(edge_hbm, h_hbm, p0_hbm, p1_hbm,
                    acc_sp, slots, *bufsem):
        bufs = bufsem[:_NBUF]
        gsem = bufsem[_NBUF:2 * _NBUF]
        ssem = bufsem[2 * _NBUF:3 * _NBUF]
        isem = bufsem[3 * _NBUF:]
        c = lax.axis_index("c")
        s = lax.axis_index("s")
        wid = c * _NS + s
        ebase = wid * EPW

        def load_idx(j, q):
            pltpu.async_copy(edge_hbm.at[pl.ds(ebase + j * _CH, _CH)],
                             slots.at[q, 0], isem[q])
            pltpu.async_copy(edge_hbm.at[pl.ds(E + ebase + j * _CH, _CH)],
                             slots.at[q, 1], isem[q])

        def wait_idx(q):
            pltpu.make_async_copy(edge_hbm.at[pl.ds(0, _CH)],
                                  slots.at[q, 0], isem[q]).wait()
            pltpu.make_async_copy(edge_hbm.at[pl.ds(0, _CH)],
                                  slots.at[q, 1], isem[q]).wait()

        def gather(b, q):
            pltpu.async_copy(h_hbm.at[slots.at[q, 0]], bufs[b], gsem[b])

        def scatter(b, q):
            pltpu.async_copy(bufs[b], acc_sp.at[slots.at[q, 1]], ssem[b],
                             add=True)

        def wait_g(b):
            pltpu.make_async_copy(h_hbm.at[slots.at[0, 0]], bufs[b],
                                  gsem[b]).wait()

        def wait_s(b):
            pltpu.make_async_copy(bufs[b], acc_sp.at[slots.at[0, 1]],
                                  ssem[b]).wait()

        for q in range(_NSLOT):
            load_idx(q, q)

        nchunks = jnp.maximum(
            0, jnp.minimum(SPT, jnp.int32(N) - s * SPT)) // _CH

        @pl.when(c == 0)
        def _ih():
            def hcp(t, carry):
                r0 = s * SPT + t * _CH
                pltpu.sync_copy(h_hbm.at[pl.ds(r0, _CH)],
                                acc_sp.at[pl.ds(r0, _CH)])
                return carry

            lax.fori_loop(0, nchunks, hcp, 0)

        @pl.when(c == 1)
        def _iz():
            def zb(i, carry):
                def zl(k, carry2):
                    bufs[0][i, pl.ds(k * _LANES, _LANES)] = jnp.zeros(
                        (_LANES,), jnp.float32)
                    return carry2

                lax.fori_loop(0, D // _LANES, zl, 0)
                return carry

            lax.fori_loop(0, _CH, zb, 0)

            def zcp(t, carry):
                pltpu.sync_copy(bufs[0],
                                acc_sp.at[pl.ds(s * SPT + t * _CH, _CH)])
                return carry

            lax.fori_loop(0, nchunks, zcp, 0)

        for b in range(_NBUF):
            wait_idx(2 * b)
            gather(b, 2 * b)
        plsc.subcore_barrier()

        def body(t, carry):
            for u in range(2 * _NBUF):
                b = u % _NBUF
                par = u // _NBUF
                q = 2 * b + par
                qn = 2 * b + (1 - par)
                j = 2 * _NBUF * t + u

                @pl.when(j < ITERS)
                def _(b=b, j=j, q=q, qn=qn):
                    wait_g(b)
                    scatter(b, q)

                    @pl.when(j + _NBUF < ITERS)
                    def _():
                        wait_s(b)

                        @pl.when(j + 2 * _NBUF < ITERS)
                        def _():
                            load_idx(j + 2 * _NBUF, q)

                        wait_idx(qn)
                        gather(b, qn)

            return carry

        lax.fori_loop(0, (ITERS + 2 * _NBUF - 1) // (2 * _NBUF), body, 0)
        for b in range(_NBUF):
            wait_s(b)

        plsc.subcore_barrier()

        @pl.when(c == 0)
        def _w0():
            pltpu.sync_copy(acc_sp.at[pl.ds(s * SPT, SPT)],
                            p0_hbm.at[pl.ds(s * SPT, SPT)])

        @pl.when(c == 1)
        def _w1():
            pltpu.sync_copy(acc_sp.at[pl.ds(s * SPT, SPT)],
                            p1_hbm.at[pl.ds(s * SPT, SPT)])

    return edge_kernel


def _make_final_kernel(N, NPAD, D):
    BLK = 2000
    GRID = N // BLK

    def body(p0_ref, p1_ref, deg_ref, out_ref):
        deg = deg_ref[:, 0:1] + deg_ref[:, 1:2]
        dis = lax.rsqrt(deg + 1.0)
        z = (p0_ref[...] + p1_ref[...]) * dis
        out_ref[...] = jnp.where(z > 0, 2.0 * z, z)

    return pl.pallas_call(
        body,
        grid=(GRID,),
        in_specs=[
            pl.BlockSpec((BLK, D), lambda i: (i, 0)),
            pl.BlockSpec((BLK, D), lambda i: (i, 0)),
            pl.BlockSpec((BLK, _NC), lambda i: (i, 0)),
        ],
        out_specs=pl.BlockSpec((BLK, D), lambda i: (i, 0)),
        out_shape=jax.ShapeDtypeStruct((N, D), jnp.float32),
    )


def kernel(x, edge_index, edge_attr, W):
    N, D = x.shape
    E = edge_index.shape[1]
    NPAD = ((N + 1023) // 1024) * 1024

    eflat = edge_index.reshape(2 * E)
    parts = _make_deg_kernel(E, NPAD)(eflat)
    parts_t = parts.T
    h = _make_mm_kernel(N, NPAD, D)(x, W, parts_t)
    p0, p1 = _make_edge_kernel(E, N, NPAD, D)(eflat, h)
    return _make_final_kernel(N, NPAD, D)(p0, p1, parts_t)
